# Initial kernel scaffold; baseline (speedup 1.0000x reference)
import jax
import jax.numpy as jnp
from jax.experimental import pallas as pl


def kernel(x, edge_index, edge_weigth, batch, W1, b1, W2, b2, W3, b3, W4, b4, Wf1, bf1, Wf2, bf2, Wf3, bf3):
    raise NotImplementedError("write your pallas kernel here")


# speedup vs baseline: 1.4900x; 1.4900x over previous
import functools

import jax
import jax.numpy as jnp
from jax import lax
from jax.experimental import pallas as pl
from jax.experimental.pallas import tpu as pltpu
from jax.experimental.pallas import tpu_sc as plsc

N = 100000
E = 1600000
NGRAPH = 32
SLOPE = 0.01

NC, NS, LANES = 2, 16, 16
CH = 80
NCHUNK = E // CH
CPT = NCHUNK // NS
HALF = 50048
N_PAD = NC * HALF
ACC_ROWS = 50176
ZR = ACC_ROWS // NS
DR = HALF // NS

R = 4000
NB = N // R


def _leaky(v):
    return jnp.where(v >= 0, v, SLOPE * v)


@functools.lru_cache(None)
def _make_agg(G):
    mesh = plsc.VectorSubcoreMesh(core_axis_name="c", subcore_axis_name="s",
                                  num_cores=NC, num_subcores=NS)
    out_type = [jax.ShapeDtypeStruct((N_PAD, LANES), jnp.float32)
                for _ in range(G)]
    scratch = [
        pltpu.VMEM((CH,), jnp.int32),
        pltpu.VMEM((CH,), jnp.int32),
        pltpu.VMEM((CH,), jnp.int32),
        pltpu.VMEM((CH,), jnp.float32),
        pltpu.VMEM((CH, LANES), jnp.float32),
        pltpu.VMEM((ZR, LANES), jnp.float32),
        pltpu.VMEM_SHARED((ACC_ROWS, LANES), jnp.float32),
        pltpu.SemaphoreType.DMA,
    ]

    def body(*refs):
        h_refs = refs[:G]
        src2, dst2, ew2 = refs[G:G + 3]
        outs = refs[G + 3:G + 3 + G]
        src_v, dst_v, loc_v, ew_v, rows_v, zbuf, acc, gsem = refs[G + 3 + G:]

        c = lax.axis_index("c")
        s = lax.axis_index("s")
        lo = c * HALF
        chunk0 = s * CPT

        def zfill(i, _):
            zbuf[i, :] = jnp.zeros((LANES,), jnp.float32)
            return 0
        lax.fori_loop(0, ZR, zfill, 0)

        for g in range(G):
            pltpu.sync_copy(zbuf, acc.at[pl.ds(s * ZR, ZR)])
            plsc.subcore_barrier()

            h_ref = h_refs[g]
            out_ref = outs[g]

            def chunk_body(j, _):
                ch = chunk0 + j
                pltpu.sync_copy(src2.at[ch], src_v)
                pltpu.sync_copy(dst2.at[ch], dst_v)
                pltpu.sync_copy(ew2.at[ch], ew_v)
                pltpu.async_copy(h_ref.at[src_v], rows_v, gsem).wait()

                def scale(q, _):
                    sl = pl.ds(q * LANES, LANES)
                    dv = dst_v[sl] - lo
                    ok = (dv >= 0) & (dv < HALF)
                    loc_v[sl] = jnp.where(ok, dv, HALF)
                    ewv = ew_v[sl]
                    for jj in range(LANES):
                        i = q * LANES + jj
                        rows_v[i, :] = rows_v[i, :] * ewv[jj]
                    return 0
                lax.fori_loop(0, CH // LANES, scale, 0)
                pltpu.sync_copy(rows_v, acc.at[loc_v], add=True)
                return 0
            lax.fori_loop(0, CPT, chunk_body, 0)
            plsc.subcore_barrier()

            r = s * DR
            pltpu.sync_copy(acc.at[pl.ds(r, DR)], zbuf.at[pl.ds(0, DR)])
            pltpu.sync_copy(zbuf.at[pl.ds(0, DR)],
                            out_ref.at[pl.ds(lo + r, DR)])
            if g + 1 < G:
                def zfill2(i, _):
                    zbuf[i, :] = jnp.zeros((LANES,), jnp.float32)
                    return 0
                lax.fori_loop(0, DR, zfill2, 0)
                plsc.subcore_barrier()

    return pl.kernel(body, out_type=out_type, mesh=mesh,
                     scratch_types=scratch,
                     compiler_params=pltpu.CompilerParams(
                         use_tc_tiling_on_sc=False))


def _mid_body(G_in, G_out, *refs):
    p_refs = refs[:G_in]
    w_ref, b_ref = refs[G_in], refs[G_in + 1]
    out_refs = refs[G_in + 2:]
    parts = [p_refs[g][...] for g in range(G_in)]
    h = jnp.concatenate(parts, axis=-1) if G_in > 1 else parts[0]
    z = _leaky(jnp.dot(h, w_ref[...], preferred_element_type=jnp.float32)
               + b_ref[...])
    for g in range(G_out):
        out_refs[g][...] = z[:, g * 16:(g + 1) * 16]


def _mid(ps, wT, b2d, G_in, G_out):
    return pl.pallas_call(
        functools.partial(_mid_body, G_in, G_out),
        grid=(NB,),
        in_specs=(
            [pl.BlockSpec((R, 16), lambda i: (i, 0))] * G_in
            + [pl.BlockSpec((16 * G_in, 16 * G_out), lambda i: (0, 0)),
               pl.BlockSpec((1, 16 * G_out), lambda i: (0, 0))]),
        out_specs=[pl.BlockSpec((R, 16), lambda i: (i, 0))] * G_out,
        out_shape=[jax.ShapeDtypeStruct((N, 16), jnp.float32)] * G_out,
    )(*ps, wT, b2d)


def _final_body(*refs):
    (p0, p1, p2, p3, w4, b4, batch_ref,
     wf1, bf1, wf2, bf2, wf3, bf3, out_ref, acc_ref) = refs
    i = pl.program_id(0)
    h = jnp.concatenate([p[...] for p in (p0, p1, p2, p3)], axis=-1)
    conv = _leaky(jnp.dot(h, w4[...], preferred_element_type=jnp.float32)
                  + b4[...])
    bn = _leaky(conv)
    seg = batch_ref[0, 0, :]
    one = (lax.broadcasted_iota(jnp.int32, (NGRAPH, R), 0)
           == seg[None, :]).astype(jnp.float32)
    part = jnp.dot(one, bn, preferred_element_type=jnp.float32)

    @pl.when(i == 0)
    def _():
        acc_ref[...] = part

    @pl.when(i > 0)
    def _():
        acc_ref[...] = acc_ref[...] + part

    @pl.when(i == NB - 1)
    def _():
        o = _leaky(jnp.dot(acc_ref[...], wf1[...],
                           preferred_element_type=jnp.float32) + bf1[...])
        o = _leaky(jnp.dot(o, wf2[...],
                           preferred_element_type=jnp.float32) + bf2[...])
        o = _leaky(jnp.dot(o, wf3[...],
                           preferred_element_type=jnp.float32) + bf3[...])
        out_ref[...] = o


def _final(ps, w4T, b4r, batch3, wf1T, bf1r, wf2T, bf2r, wf3T, bf3r):
    return pl.pallas_call(
        _final_body,
        grid=(NB,),
        in_specs=(
            [pl.BlockSpec((R, 16), lambda i: (i, 0))] * 4
            + [pl.BlockSpec((64, 50), lambda i: (0, 0)),
               pl.BlockSpec((1, 50), lambda i: (0, 0)),
               pl.BlockSpec((1, 1, R), lambda i: (i, 0, 0)),
               pl.BlockSpec((50, 30), lambda i: (0, 0)),
               pl.BlockSpec((1, 30), lambda i: (0, 0)),
               pl.BlockSpec((30, 20), lambda i: (0, 0)),
               pl.BlockSpec((1, 20), lambda i: (0, 0)),
               pl.BlockSpec((20, 2), lambda i: (0, 0)),
               pl.BlockSpec((1, 2), lambda i: (0, 0))]),
        out_specs=pl.BlockSpec((NGRAPH, 2), lambda i: (0, 0)),
        out_shape=jax.ShapeDtypeStruct((NGRAPH, 2), jnp.float32),
        scratch_shapes=[pltpu.VMEM((NGRAPH, 50), jnp.float32)],
    )(*ps, w4T, b4r, batch3, wf1T, bf1r, wf2T, bf2r, wf3T, bf3r)


def kernel(x, edge_index, edge_weigth, batch,
           W1, b1, W2, b2, W3, b3, W4, b4,
           Wf1, bf1, Wf2, bf2, Wf3, bf3):
    src2 = edge_index[0].astype(jnp.int32).reshape(NCHUNK, CH)
    dst2 = edge_index[1].astype(jnp.int32).reshape(NCHUNK, CH)
    ew2 = edge_weigth.reshape(NCHUNK, CH)
    xpad = jnp.pad(x, ((0, 0), (0, 16 - x.shape[1])))
    batch3 = batch.astype(jnp.int32).reshape(NB, 1, R)

    w1T = jnp.zeros((16, 16), jnp.float32).at[:6, :].set(W1.T)
    agg1 = _make_agg(1)
    agg2 = _make_agg(2)
    agg4 = _make_agg(4)

    p1 = agg1(xpad, src2, dst2, ew2)
    h1 = _mid(list(p1), w1T, b1.reshape(1, 16), 1, 1)[0]
    p2 = agg1(h1, src2, dst2, ew2)
    h2g = _mid(list(p2), W2.T, b2.reshape(1, 32), 1, 2)
    p3 = agg2(*h2g, src2, dst2, ew2)
    h3g = _mid(list(p3), W3.T, b3.reshape(1, 64), 2, 4)
    p4 = agg4(*h3g, src2, dst2, ew2)

    return _final(list(p4), W4.T, b4.reshape(1, 50), batch3,
                  Wf1.T, bf1.reshape(1, 30),
                  Wf2.T, bf2.reshape(1, 20),
                  Wf3.T, bf3.reshape(1, 2))
